# baseline (speedup 1.0000x reference)
import jax
import jax.numpy as jnp
from jax import lax
from jax.experimental import pallas as pl
from jax.experimental.pallas import tpu as pltpu
from jax.experimental.pallas import tpu_sc as plsc

VOCAB = 1000000
EMBED_DIM = 64
BATCH = 16384
TITLE_LEN = 20

NUM_CORES = 2
NUM_SUBCORES = 16
LANES = 16
NUM_WORKERS = NUM_CORES * NUM_SUBCORES

LINE_W = 128
B_PER_W = BATCH // NUM_WORKERS
TOK_PER_W = B_PER_W * TITLE_LEN
IDX_ROW = 128
IDX_ROWS_PER_W = TOK_PER_W // IDX_ROW
CB = 32
TOK_PER_CHUNK = CB * TITLE_LEN
GATHERS_PER_CHUNK = TOK_PER_CHUNK // IDX_ROW
NSTEPS = B_PER_W // CB
D_SLICES = EMBED_DIM // LANES

TCHUNK = 4096
H = TCHUNK // 2
NBLK = pl.cdiv(VOCAB, TCHUNK)
NROWS = NBLK * TCHUNK


def _t_body(tT_ref, out_ref):
    x = tT_ref[...].astype(jnp.bfloat16)
    c = jnp.concatenate([x[:, 0:H], x[:, H:TCHUNK]], axis=0)
    kk = lax.broadcasted_iota(jnp.int32, (LINE_W, LINE_W), 0)
    ll = lax.broadcasted_iota(jnp.int32, (LINE_W, LINE_W), 1)
    lm = ll % EMBED_DIM
    sigma = ((ll // EMBED_DIM) * EMBED_DIM + (lm // 32) * 32 +
             (lm % 2) * 16 + (lm % 32) // 2)
    p = (kk == sigma).astype(jnp.bfloat16)
    dn = (((0,), (0,)), ((), ()))
    y = lax.dot_general(c, p, dn, preferred_element_type=jnp.float32)
    out_ref[...] = y.astype(jnp.bfloat16).reshape(TCHUNK * EMBED_DIM)


def _to_packed_rows(tableT):
    return pl.pallas_call(
        _t_body,
        grid=(NBLK,),
        in_specs=[pl.BlockSpec((EMBED_DIM, TCHUNK), lambda i: (0, i))],
        out_specs=pl.BlockSpec((TCHUNK * EMBED_DIM,), lambda i: (i,)),
        out_shape=jax.ShapeDtypeStruct((NROWS * EMBED_DIM,), jnp.bfloat16),
    )(tableT)


def _body(idx_hbm, table_hbm, out_hbm, idx_v, rows_v, out_v, gsem0, gsem1):
    wid = lax.axis_index("s") * NUM_CORES + lax.axis_index("c")

    pltpu.sync_copy(idx_hbm.at[pl.ds(wid * IDX_ROWS_PER_W, IDX_ROWS_PER_W), :],
                    idx_v)

    gsems = (gsem0, gsem1)

    def start_chunk(s, buf):
        for g in range(GATHERS_PER_CHUNK):
            pltpu.async_copy(
                table_hbm.at[idx_v.at[s * GATHERS_PER_CHUNK + g]],
                rows_v.at[buf, pl.ds(g * IDX_ROW, IDX_ROW), :],
                gsems[buf])

    def wait_chunk(buf):
        for g in range(GATHERS_PER_CHUNK):
            pltpu.make_async_copy(
                table_hbm.at[idx_v.at[0]],
                rows_v.at[buf, pl.ds(g * IDX_ROW, IDX_ROW), :],
                gsems[buf]).wait()

    def reduce_chunk(s, buf):
        rows = rows_v.at[buf]
        inv_l = jnp.float32(1.0 / TITLE_LEN)

        def body(b, carry):
            tok = b * TITLE_LEN
            vals = [[] for _ in range(D_SLICES)]
            for t in range(TITLE_LEN):
                for g2 in range(2):
                    ab = rows[tok + t, pl.ds(g2 * 32, 32)]
                    a, bb = plsc.unpack(ab, format=plsc.PackFormat.INTERLEAVED)
                    vals[2 * g2].append(a)
                    vals[2 * g2 + 1].append(bb)
            for d in range(D_SLICES):
                vs = vals[d]
                while len(vs) > 1:
                    nxt = [vs[i] + vs[i + 1] for i in range(0, len(vs) - 1, 2)]
                    if len(vs) % 2:
                        nxt.append(vs[-1])
                    vs = nxt
                out_v[s * CB + b, pl.ds(d * LANES, LANES)] = vs[0] * inv_l
            return carry

        lax.fori_loop(0, CB, body, 0)

    start_chunk(0, 0)
    for s in range(NSTEPS):
        buf = s % 2
        if s + 1 < NSTEPS:
            start_chunk(s + 1, 1 - buf)
        wait_chunk(buf)
        reduce_chunk(s, buf)

    pltpu.sync_copy(out_v, out_hbm.at[pl.ds(wid * B_PER_W, B_PER_W), :])


@jax.jit
def kernel(title, word_emb_table):
    t32 = title.astype(jnp.int32)
    packed = (t32 // TCHUNK) * TCHUNK + ((t32 % H) << 1) + ((t32 // H) & 1)
    idx2d = packed.reshape(NUM_WORKERS * IDX_ROWS_PER_W, IDX_ROW)
    table_bf = _to_packed_rows(word_emb_table.T).reshape(NROWS, EMBED_DIM)
    mesh = plsc.VectorSubcoreMesh(core_axis_name="c", subcore_axis_name="s")
    f = pl.kernel(
        _body,
        out_type=jax.ShapeDtypeStruct((BATCH, EMBED_DIM), jnp.float32),
        mesh=mesh,
        scratch_types=[
            pltpu.VMEM((IDX_ROWS_PER_W, IDX_ROW), jnp.int32),
            pltpu.VMEM((2, TOK_PER_CHUNK, EMBED_DIM), jnp.bfloat16),
            pltpu.VMEM((B_PER_W, EMBED_DIM), jnp.float32),
            pltpu.SemaphoreType.DMA,
            pltpu.SemaphoreType.DMA,
        ],
        compiler_params=pltpu.CompilerParams(use_tc_tiling_on_sc=False,
                                             needs_layout_passes=False),
    )
    return f(idx2d, table_bf)


# speedup vs baseline: 2.0202x; 2.0202x over previous
import jax
import jax.numpy as jnp
from jax import lax
from jax.experimental import pallas as pl
from jax.experimental.pallas import tpu as pltpu
from jax.experimental.pallas import tpu_sc as plsc

VOCAB = 1000000
EMBED_DIM = 64
BATCH = 16384
TITLE_LEN = 20

NUM_CORES = 2
NUM_SUBCORES = 16
LANES = 16
NUM_WORKERS = NUM_CORES * NUM_SUBCORES

LINE_W = 128
B_PER_W = BATCH // NUM_WORKERS
TOK_PER_W = B_PER_W * TITLE_LEN
GATHER_N = 64
CB = 16
TOK_PER_CHUNK = CB * TITLE_LEN
GATHERS_PER_CHUNK = TOK_PER_CHUNK // GATHER_N
NSTEPS = B_PER_W // CB
D_SLICES = EMBED_DIM // LANES

TCHUNK = 4096
H = TCHUNK // 2
NBLK = pl.cdiv(VOCAB, TCHUNK)
NLINES = NBLK * H


def _t_body(tT_ref, out_ref):
    x = tT_ref[...].astype(jnp.bfloat16)
    c = jnp.concatenate([x[:, 0:H], x[:, H:TCHUNK]], axis=0)
    eye = (lax.broadcasted_iota(jnp.int32, (LINE_W, LINE_W), 0) ==
           lax.broadcasted_iota(jnp.int32, (LINE_W, LINE_W), 1)
           ).astype(jnp.bfloat16)
    dn = (((0,), (0,)), ((), ()))
    out_ref[...] = lax.dot_general(c, eye, dn,
                                   preferred_element_type=jnp.float32)


def _to_packed_lines(tableT):
    return pl.pallas_call(
        _t_body,
        grid=(NBLK,),
        in_specs=[pl.BlockSpec((EMBED_DIM, TCHUNK), lambda i: (0, i))],
        out_specs=pl.BlockSpec((H, LINE_W), lambda i: (i, 0)),
        out_shape=jax.ShapeDtypeStruct((NLINES, LINE_W), jnp.float32),
    )(tableT)


def _body(titleT_hbm, table_hbm, out_hbm, tv_v, idx_v, rows_v, out_v,
          gsem0, gsem1):
    wid = lax.axis_index("s") * NUM_CORES + lax.axis_index("c")

    pltpu.sync_copy(titleT_hbm.at[:, pl.ds(wid * B_PER_W, B_PER_W)], tv_v)

    lane20 = lax.iota(jnp.int32, LANES) * TITLE_LEN

    def idx_body(g, carry):
        t = g // (B_PER_W // LANES)
        c0 = (g % (B_PER_W // LANES)) * LANES
        v = tv_v[t, pl.ds(c0, LANES)]
        packed = (((v >> 12) << 12) | ((v & (H - 1)) << 1) | ((v >> 11) & 1))
        pos = lane20 + (c0 * TITLE_LEN + t)
        plsc.store_scatter(idx_v, [pos], packed)
        return carry

    lax.fori_loop(0, TOK_PER_W // LANES, idx_body, 0)

    gsems = (gsem0, gsem1)

    def start_chunk(s, buf):
        for g in range(GATHERS_PER_CHUNK):
            pltpu.async_copy(
                table_hbm.at[idx_v.at[pl.ds(s * TOK_PER_CHUNK + g * GATHER_N,
                                            GATHER_N)]],
                rows_v.at[buf, pl.ds(g * GATHER_N, GATHER_N), :],
                gsems[buf])

    def wait_chunk(buf):
        for g in range(GATHERS_PER_CHUNK):
            pltpu.make_async_copy(
                table_hbm.at[idx_v.at[pl.ds(0, GATHER_N)]],
                rows_v.at[buf, pl.ds(g * GATHER_N, GATHER_N), :],
                gsems[buf]).wait()

    def reduce_chunk(s, buf):
        rows = rows_v.at[buf]
        inv_l = jnp.float32(1.0 / TITLE_LEN)

        def body(b, carry):
            tok = b * TITLE_LEN
            for d in range(D_SLICES):
                sl = pl.ds(d * LANES, LANES)
                vals = [rows[tok + t, sl] for t in range(TITLE_LEN)]
                while len(vals) > 1:
                    nxt = [vals[i] + vals[i + 1] for i in range(0, len(vals) - 1, 2)]
                    if len(vals) % 2:
                        nxt.append(vals[-1])
                    vals = nxt
                out_v[s * CB + b, sl] = vals[0] * inv_l
            return carry

        lax.fori_loop(0, CB, body, 0)

    start_chunk(0, 0)
    start_chunk(1, 1)
    wait_chunk(0)
    reduce_chunk(0, 0)
    start_chunk(2, 0)
    wait_chunk(1)
    reduce_chunk(1, 1)

    def outer(k, carry):
        s0 = 2 * k
        start_chunk(s0 + 1, 1)
        wait_chunk(0)
        reduce_chunk(s0, 0)

        @pl.when(k < NSTEPS // 2 - 1)
        def _():
            start_chunk(s0 + 2, 0)
        wait_chunk(1)
        reduce_chunk(s0 + 1, 1)
        return carry

    lax.fori_loop(1, NSTEPS // 2, outer, 0)

    pltpu.sync_copy(out_v, out_hbm.at[pl.ds(wid * B_PER_W, B_PER_W), :])


@jax.jit
def kernel(title, word_emb_table):
    titleT = jnp.transpose(title.astype(jnp.int32))
    table_rm = _to_packed_lines(word_emb_table.T).reshape(2 * NLINES, EMBED_DIM)
    mesh = plsc.VectorSubcoreMesh(core_axis_name="c", subcore_axis_name="s")
    f = pl.kernel(
        _body,
        out_type=jax.ShapeDtypeStruct((BATCH, EMBED_DIM), jnp.float32),
        mesh=mesh,
        scratch_types=[
            pltpu.VMEM((TITLE_LEN, B_PER_W), jnp.int32),
            pltpu.VMEM((TOK_PER_W,), jnp.int32),
            pltpu.VMEM((2, TOK_PER_CHUNK, EMBED_DIM), jnp.float32),
            pltpu.VMEM((B_PER_W, EMBED_DIM), jnp.float32),
            pltpu.SemaphoreType.DMA,
            pltpu.SemaphoreType.DMA,
        ],
        compiler_params=pltpu.CompilerParams(use_tc_tiling_on_sc=False,
                                             needs_layout_passes=False),
    )
    return f(titleT, table_rm)
